# Initial kernel scaffold; baseline (speedup 1.0000x reference)
import functools

import jax
import jax.numpy as jnp
from jax import lax
from jax.experimental import pallas as pl
from jax.experimental.pallas import tpu as pltpu
from jax.experimental.pallas import tpu_sc as plsc

N = 10000
E = 320000
D_IN = 128
D_H = 64

NC = 2
NS = 16
NW = NC * NS
CHUNK = 128
NCH = 80
EPW = NCH * CHUNK
EPAD = NW * EPW
NPAD = 10240
RPS = NPAD // NS

_mesh = plsc.VectorSubcoreMesh(
    core_axis_name="c", subcore_axis_name="s", num_cores=NC, num_subcores=NS
)


@functools.partial(
    pl.kernel,
    out_type=jax.ShapeDtypeStruct((NC, NPAD, D_H), jnp.float32),
    mesh=_mesh,
    scratch_types=[
        pltpu.VMEM((NCH, CHUNK), jnp.int32),
        pltpu.VMEM((NCH, CHUNK), jnp.int32),
        pltpu.VMEM((CHUNK, D_H), jnp.float32),
        pltpu.VMEM((CHUNK, D_H), jnp.float32),
        pltpu.VMEM_SHARED((NPAD, D_H), jnp.float32),
        pltpu.SemaphoreType.DMA,
        pltpu.SemaphoreType.DMA,
    ],
)
def _sc_segment_sum(y_hbm, src_hbm, dst_hbm, zero_hbm, out_hbm,
                    src_v, dst_v, buf_a, buf_b, acc, sem_a, sem_b):
    cid = lax.axis_index("c")
    sid = lax.axis_index("s")
    wid = cid * NS + sid

    rows = pl.ds(sid * RPS, RPS)
    pltpu.sync_copy(zero_hbm.at[rows], acc.at[rows])
    pltpu.sync_copy(src_hbm.at[wid], src_v)
    pltpu.sync_copy(dst_hbm.at[wid], dst_v)
    plsc.subcore_barrier()

    def gather(j, buf, sem):
        return pltpu.async_copy(y_hbm.at[src_v.at[j]], buf, sem)

    def scat_add(j, buf):
        pltpu.sync_copy(buf, acc.at[dst_v.at[j]], add=True)

    def wait(j, buf, sem):
        pltpu.make_async_copy(y_hbm.at[src_v.at[j]], buf, sem).wait()

    gather(0, buf_a, sem_a)

    @pl.loop(0, NCH - 2, step=2)
    def _(j):
        gather(j + 1, buf_b, sem_b)
        wait(j, buf_a, sem_a)
        scat_add(j, buf_a)
        gather(j + 2, buf_a, sem_a)
        wait(j + 1, buf_b, sem_b)
        scat_add(j + 1, buf_b)

    gather(NCH - 1, buf_b, sem_b)
    wait(NCH - 2, buf_a, sem_a)
    scat_add(NCH - 2, buf_a)
    wait(NCH - 1, buf_b, sem_b)
    scat_add(NCH - 1, buf_b)

    plsc.subcore_barrier()
    pltpu.sync_copy(acc.at[rows], out_hbm.at[cid, rows])


def _mm1_body(x_ref, w_ref, o_ref):
    o_ref[...] = jnp.dot(x_ref[...], w_ref[...], preferred_element_type=jnp.float32)


_mm1 = pl.pallas_call(
    _mm1_body, out_shape=jax.ShapeDtypeStruct((NPAD, D_H), jnp.float32)
)


def _mlp2_body(y_ref, par_ref, b1_ref, w2_ref, b2_ref, o_ref):
    h = y_ref[...] + par_ref[0] + par_ref[1] + b1_ref[...]
    h = jnp.maximum(h, 0.0)
    o_ref[...] = jnp.dot(h, w2_ref[...], preferred_element_type=jnp.float32) + b2_ref[...]


_mlp2 = pl.pallas_call(
    _mlp2_body, out_shape=jax.ShapeDtypeStruct((NPAD, D_H), jnp.float32)
)


def kernel(x, edge_index, W1, b1, W2, b2):
    x = x.astype(jnp.float32)
    ei = edge_index.astype(jnp.int32)
    x_pad = jnp.pad(x, ((0, NPAD - N), (0, 0)))
    fill = N + (jnp.arange(EPAD - E, dtype=jnp.int32) % (NPAD - N))
    src = jnp.concatenate([ei[0], fill]).reshape(NW, NCH, CHUNK)
    dst = jnp.concatenate([ei[1], fill]).reshape(NW, NCH, CHUNK)

    y = _mm1(x_pad, W1)
    zeros = jnp.zeros((NPAD, D_H), jnp.float32)
    partials = _sc_segment_sum(y, src, dst, zeros)
    out = _mlp2(y, partials, b1.reshape(1, D_H), W2, b2.reshape(1, D_H))
    return out[:N]


# speedup vs baseline: 13.9673x; 13.9673x over previous
import functools

import jax
import jax.numpy as jnp
from jax import lax
from jax.experimental import pallas as pl
from jax.experimental.pallas import tpu as pltpu
from jax.experimental.pallas import tpu_sc as plsc

N = 10000
E = 320000
D_IN = 128
D_H = 64

NC = 2
NS = 16
NW = NC * NS
CHUNK = 128
NCH = 80
EPW = NCH * CHUNK
EPAD = NW * EPW
NPAD = 10240
RPS = NPAD // NS

_mesh = plsc.VectorSubcoreMesh(
    core_axis_name="c", subcore_axis_name="s", num_cores=NC, num_subcores=NS
)


@functools.partial(
    pl.kernel,
    out_type=jax.ShapeDtypeStruct((NC, NPAD, D_H), jnp.float32),
    mesh=_mesh,
    scratch_types=[
        pltpu.VMEM((NCH, CHUNK), jnp.int32),
        pltpu.VMEM((NCH, CHUNK), jnp.int32),
        pltpu.VMEM((CHUNK, D_H), jnp.float32),
        pltpu.VMEM((CHUNK, D_H), jnp.float32),
        pltpu.VMEM_SHARED((NPAD, D_H), jnp.float32),
        pltpu.SemaphoreType.DMA,
        pltpu.SemaphoreType.DMA,
    ],
    compiler_params=pltpu.CompilerParams(use_tc_tiling_on_sc=False),
)
def _sc_segment_sum(y_hbm, src_hbm, dst_hbm, zero_hbm, out_hbm,
                    src_v, dst_v, buf_a, buf_b, acc, sem_a, sem_b):
    cid = lax.axis_index("c")
    sid = lax.axis_index("s")
    wid = cid * NS + sid

    rows = pl.ds(sid * RPS, RPS)
    pltpu.sync_copy(zero_hbm.at[rows], acc.at[rows])
    pltpu.sync_copy(src_hbm.at[wid], src_v)
    pltpu.sync_copy(dst_hbm.at[wid], dst_v)
    plsc.subcore_barrier()

    def gather(j, buf, sem):
        return pltpu.async_copy(y_hbm.at[src_v.at[j]], buf, sem)

    def scat_add(j, buf):
        pltpu.sync_copy(buf, acc.at[dst_v.at[j]], add=True)

    def wait(j, buf, sem):
        pltpu.make_async_copy(y_hbm.at[src_v.at[j]], buf, sem).wait()

    gather(0, buf_a, sem_a)

    @pl.loop(0, NCH - 2, step=2)
    def _(j):
        gather(j + 1, buf_b, sem_b)
        wait(j, buf_a, sem_a)
        scat_add(j, buf_a)
        gather(j + 2, buf_a, sem_a)
        wait(j + 1, buf_b, sem_b)
        scat_add(j + 1, buf_b)

    gather(NCH - 1, buf_b, sem_b)
    wait(NCH - 2, buf_a, sem_a)
    scat_add(NCH - 2, buf_a)
    wait(NCH - 1, buf_b, sem_b)
    scat_add(NCH - 1, buf_b)

    plsc.subcore_barrier()
    pltpu.sync_copy(acc.at[rows], out_hbm.at[cid, rows])


def _mm1_body(x_ref, w_ref, o_ref):
    o_ref[...] = jnp.dot(x_ref[...], w_ref[...], preferred_element_type=jnp.float32)


_mm1 = pl.pallas_call(
    _mm1_body, out_shape=jax.ShapeDtypeStruct((NPAD, D_H), jnp.float32)
)


def _mlp2_body(y_ref, par_ref, b1_ref, w2_ref, b2_ref, o_ref):
    h = y_ref[...] + par_ref[0] + par_ref[1] + b1_ref[...]
    h = jnp.maximum(h, 0.0)
    o_ref[...] = jnp.dot(h, w2_ref[...], preferred_element_type=jnp.float32) + b2_ref[...]


_mlp2 = pl.pallas_call(
    _mlp2_body, out_shape=jax.ShapeDtypeStruct((NPAD, D_H), jnp.float32)
)


def kernel(x, edge_index, W1, b1, W2, b2):
    x = x.astype(jnp.float32)
    ei = edge_index.astype(jnp.int32)
    x_pad = jnp.pad(x, ((0, NPAD - N), (0, 0)))
    fill = N + (jnp.arange(EPAD - E, dtype=jnp.int32) % (NPAD - N))
    src = jnp.concatenate([ei[0], fill]).reshape(NW, NCH, CHUNK)
    dst = jnp.concatenate([ei[1], fill]).reshape(NW, NCH, CHUNK)

    y = _mm1(x_pad, W1)
    zeros = jnp.zeros((NPAD, D_H), jnp.float32)
    partials = _sc_segment_sum(y, src, dst, zeros)
    out = _mlp2(y, partials, b1.reshape(1, D_H), W2, b2.reshape(1, D_H))
    return out[:N]
